# baseline (speedup 1.0000x reference)
import functools

import jax
import jax.numpy as jnp
from jax import lax
from jax.experimental import pallas as pl
from jax.experimental.pallas import tpu as pltpu
from jax.experimental.pallas import tpu_sc as plsc

V = 63
FEAT = 256
D = 64
B = 4096
L = 200

NC = 2
NS = 16
NW = NC * NS
BBLK = B // NW
NBUF = 2
REP = 16


def _table_body(w_ref, sym_ref, b_ref, out_ref):
    out_ref[...] = (
        jax.lax.dot_general(
            w_ref[...],
            sym_ref[...],
            (((0,), (1,)), ((), ())),
            preferred_element_type=jnp.float32,
        )
        + b_ref[...]
    )


def _make_table_t(symbols, W, b):
    sym_pad = jnp.pad(symbols, ((0, 64 - V), (0, 0)))
    return pl.pallas_call(
        _table_body,
        out_shape=jax.ShapeDtypeStruct((D, 64), jnp.float32),
    )(W, sym_pad, b.reshape(D, 1))


def _sc_gather_body(table_hbm, idx_hbm, out_hbm, table_v, idx_v, ob0, ob1, *ws):
    obufs = (ob0, ob1)
    wid = lax.axis_index("s") * NC + lax.axis_index("c")
    pltpu.sync_copy(table_hbm, table_v)
    pltpu.sync_copy(idx_hbm.at[:, wid], idx_v)
    iota = lax.iota(jnp.int32, 16)

    def write_start(li, bslot):
        pltpu.make_async_copy(
            obufs[bslot], out_hbm.at[li, :, wid, :, :], ws[bslot]
        ).start()

    def write_wait(bslot):
        pltpu.make_async_copy(
            obufs[bslot], out_hbm.at[0, :, 0, :, :], ws[bslot]
        ).wait()

    def l_step(lo, carry):
        for bslot in range(NBUF):
            li = lo * NBUF + bslot

            @pl.when(lo >= 1)
            def _():
                write_wait(bslot)

            pbases = [
                idx_v[li, pl.ds(bg * 16, 16)] * REP + iota for bg in range(8)
            ]

            @plsc.parallel_loop(0, 8, unroll=2)
            def _(tr):
                ptrs = [pb + tr * (8 * D * REP) for pb in pbases]
                for dl in range(8):
                    for bg in range(8):
                        v = plsc.load_gather(
                            table_v, [ptrs[bg] + dl * (D * REP)]
                        )
                        obufs[bslot][tr, dl, pl.ds(bg * 16, 16)] = v

            write_start(li, bslot)
        return carry

    lax.fori_loop(0, L // NBUF, l_step, 0)
    for bslot in range(NBUF):
        write_wait(bslot)


@functools.partial(jax.jit)
def kernel(QR, symbols, W, b):
    table_t = _make_table_t(symbols, W, b).reshape(-1)
    t16 = jnp.repeat(table_t, REP)
    idx = QR.T.reshape(L, NW, BBLK).astype(jnp.int32)
    mesh = plsc.VectorSubcoreMesh(core_axis_name="c", subcore_axis_name="s")
    gather = pl.kernel(
        _sc_gather_body,
        out_type=jax.ShapeDtypeStruct((L, 8, NW, 8, BBLK), jnp.float32),
        mesh=mesh,
        scratch_types=(
            [
                pltpu.VMEM((64 * D * REP,), jnp.float32),
                pltpu.VMEM((L, BBLK), jnp.int32),
                pltpu.VMEM((8, 8, BBLK), jnp.float32),
                pltpu.VMEM((8, 8, BBLK), jnp.float32),
            ]
            + [pltpu.SemaphoreType.DMA] * NBUF
        ),
        compiler_params=pltpu.CompilerParams(needs_layout_passes=False),
    )
    out5 = gather(t16, idx)
    out = (
        out5.transpose(0, 1, 3, 2, 4)
        .reshape(L, D, B)
        .transpose(2, 0, 1)
    )
    return out


# speedup vs baseline: 1.5921x; 1.5921x over previous
import functools

import jax
import jax.numpy as jnp
from jax import lax
from jax.experimental import pallas as pl
from jax.experimental.pallas import tpu as pltpu
from jax.experimental.pallas import tpu_sc as plsc

V = 63
FEAT = 256
D = 64
B = 4096
L = 200

NC = 2
NS = 16
NW = NC * NS
BBLK = B // NW
NBUF = 2
REP = 16


def _table_body(w_ref, sym_ref, b_ref, out_ref):
    out_ref[...] = (
        jax.lax.dot_general(
            w_ref[...],
            sym_ref[...],
            (((0,), (1,)), ((), ())),
            preferred_element_type=jnp.float32,
        )
        + b_ref[...]
    )


def _make_table_t(symbols, W, b):
    sym_pad = jnp.pad(symbols, ((0, 64 - V), (0, 0)))
    return pl.pallas_call(
        _table_body,
        out_shape=jax.ShapeDtypeStruct((D, 64), jnp.float32),
    )(W, sym_pad, b.reshape(D, 1))


def _sc_gather_body(table_hbm, idx_hbm, out_hbm, table_v, idx_v, ob0, ob1, *ws):
    obufs = (ob0, ob1)
    wid = lax.axis_index("s") * NC + lax.axis_index("c")
    pltpu.sync_copy(table_hbm, table_v)
    pltpu.sync_copy(idx_hbm.at[:, wid], idx_v)
    iota = lax.iota(jnp.int32, 16)

    def write_start(li, bslot):
        pltpu.make_async_copy(
            obufs[bslot], out_hbm.at[li, :, wid, :, :], ws[bslot]
        ).start()

    def write_wait(bslot):
        pltpu.make_async_copy(
            obufs[bslot], out_hbm.at[0, :, 0, :, :], ws[bslot]
        ).wait()

    def l_step(lo, carry):
        for bslot in range(NBUF):
            li = lo * NBUF + bslot

            @pl.when(lo >= 1)
            def _():
                write_wait(bslot)

            pbases = [
                idx_v[li, pl.ds(bg * 16, 16)] * REP + iota for bg in range(8)
            ]

            @plsc.parallel_loop(0, 8)
            def _(tr):
                ptrs = [pb + tr * (8 * D * REP) for pb in pbases]
                for dl in range(8):
                    for bg in range(8):
                        v = plsc.load_gather(
                            table_v, [ptrs[bg] + dl * (D * REP)]
                        )
                        obufs[bslot][tr, dl, pl.ds(bg * 16, 16)] = v

            write_start(li, bslot)
        return carry

    lax.fori_loop(0, L // NBUF, l_step, 0)
    for bslot in range(NBUF):
        write_wait(bslot)


@functools.partial(jax.jit)
def kernel(QR, symbols, W, b):
    table_t = _make_table_t(symbols, W, b).reshape(-1)
    t16 = jnp.repeat(table_t, REP)
    idx = QR.T.reshape(L, NW, BBLK).astype(jnp.int32)
    mesh = plsc.VectorSubcoreMesh(core_axis_name="c", subcore_axis_name="s")
    gather = pl.kernel(
        _sc_gather_body,
        out_type=jax.ShapeDtypeStruct((L, 8, NW, 8, BBLK), jnp.float32),
        mesh=mesh,
        scratch_types=(
            [
                pltpu.VMEM((64 * D * REP,), jnp.float32),
                pltpu.VMEM((L, BBLK), jnp.int32),
                pltpu.VMEM((8, 8, BBLK), jnp.float32),
                pltpu.VMEM((8, 8, BBLK), jnp.float32),
            ]
            + [pltpu.SemaphoreType.DMA] * NBUF
        ),
        compiler_params=pltpu.CompilerParams(needs_layout_passes=False),
    )
    out5 = gather(t16, idx)
    out = (
        out5.transpose(0, 1, 3, 2, 4)
        .reshape(L, D, B)
        .transpose(2, 0, 1)
    )
    return out
